# Initial kernel scaffold; baseline (speedup 1.0000x reference)
import jax
import jax.numpy as jnp
from jax.experimental import pallas as pl


def kernel(points):
    raise NotImplementedError("write your pallas kernel here")


# speedup vs baseline: 3.8194x; 3.8194x over previous
import jax
import jax.numpy as jnp
from jax import lax
from jax.experimental import pallas as pl
from jax.experimental.pallas import tpu as pltpu
from jax.experimental.pallas import tpu_sc as plsc

H, W = 512, 512
B = 64
P = 8192
IMG = H * W
CHUNK = 65536
NCHUNK = IMG // CHUNK
NC, NS = 2, 16
NW = NC * NS
BPW = B // NW
GROUPS = P // 16


def _sc_body(pts_hbm, out_hbm, pts_v, idx_v, acc_v):
    wid = lax.axis_index("s") * NC + lax.axis_index("c")
    lanes = lax.iota(jnp.int32, 16)
    ones = jnp.full((16,), 1.0, dtype=jnp.float32)
    zeros = jnp.zeros((16,), dtype=jnp.float32)

    for j in range(BPW):
        b = wid * BPW + j
        pltpu.sync_copy(pts_hbm.at[b], pts_v)

        def idx_body(g, _):
            base = g * 32
            y = plsc.load_gather(pts_v, [base + 2 * lanes])
            x = plsc.load_gather(pts_v, [base + 2 * lanes + 1])
            idx_v[pl.ds(g * 16, 16)] = y * W + x
            return 0

        lax.fori_loop(0, GROUPS, idx_body, 0)

        for c in range(NCHUNK):
            lo = c * CHUNK

            def zero_body(z, _):
                base = z * 128
                for u in range(8):
                    acc_v[pl.ds(base + u * 16, 16)] = zeros
                return 0

            lax.fori_loop(0, CHUNK // 128, zero_body, 0)

            def scat_body(g, _):
                v = idx_v[pl.ds(g * 16, 16)] - lo
                m = (v >= 0) & (v < CHUNK)
                safe = jnp.where(m, v, 0)
                plsc.addupdate_scatter(acc_v, [safe], ones, mask=m)
                return 0

            lax.fori_loop(0, GROUPS, scat_body, 0)

            pltpu.sync_copy(acc_v, out_hbm.at[b, pl.ds(lo, CHUNK)])


def kernel(points):
    pts = points.reshape(B, P * 2)
    mesh = plsc.VectorSubcoreMesh(core_axis_name="c", subcore_axis_name="s")
    out = pl.kernel(
        _sc_body,
        mesh=mesh,
        compiler_params=pltpu.CompilerParams(needs_layout_passes=False),
        out_type=jax.ShapeDtypeStruct((B, IMG), jnp.float32),
        scratch_types=[
            pltpu.VMEM((P * 2,), jnp.int32),
            pltpu.VMEM((P,), jnp.int32),
            pltpu.VMEM((CHUNK,), jnp.float32),
        ],
    )(pts)
    return out.reshape(B, H, W, 1)


# speedup vs baseline: 6.0598x; 1.5866x over previous
import jax
import jax.numpy as jnp
from jax import lax
from jax.experimental import pallas as pl
from jax.experimental.pallas import tpu as pltpu
from jax.experimental.pallas import tpu_sc as plsc

H, W = 512, 512
B = 64
P = 8192
IMG = H * W
ROWS = 128
CHUNK = ROWS * W
NCHUNK = H // ROWS
NC, NS = 2, 16
NW = NC * NS
BPW = B // NW
GROUPS = P // 16


def _sc_body(pts_hbm, out_hbm, pts_v, idx_v, acc_v):
    wid = lax.axis_index("s") * NC + lax.axis_index("c")
    lanes = lax.iota(jnp.int32, 16)
    ones = jnp.full((16,), 1.0, dtype=jnp.float32)
    zeros = jnp.zeros((16,), dtype=jnp.float32)

    for j in range(BPW):
        b = wid * BPW + j
        pltpu.sync_copy(pts_hbm.at[b], pts_v)

        def idx_body(g, _):
            base = g * 32
            y = plsc.load_gather(pts_v, [base + 2 * lanes])
            x = plsc.load_gather(pts_v, [base + 2 * lanes + 1])
            idx_v[pl.ds(g * 16, 16)] = y * W + x
            return 0

        lax.fori_loop(0, GROUPS, idx_body, 0)

        for c in range(NCHUNK):
            lo = c * CHUNK

            def zero_body(r, _):
                for u in range(W // 16):
                    acc_v[r, pl.ds(u * 16, 16)] = zeros
                return 0

            lax.fori_loop(0, ROWS, zero_body, 0)

            def scat_body(g, _):
                v = idx_v[pl.ds(g * 16, 16)] - lo
                m = (v >= 0) & (v < CHUNK)
                safe = jnp.where(m, v, 0)
                iy = lax.shift_right_logical(safe, 9)
                ix = safe & (W - 1)
                plsc.addupdate_scatter(acc_v, [iy, ix], ones, mask=m)
                return 0

            lax.fori_loop(0, GROUPS, scat_body, 0)

            pltpu.sync_copy(acc_v, out_hbm.at[b, pl.ds(c * ROWS, ROWS)])


def kernel(points):
    pts = points.reshape(B, P * 2)
    mesh = plsc.VectorSubcoreMesh(core_axis_name="c", subcore_axis_name="s")
    out = pl.kernel(
        _sc_body,
        mesh=mesh,
        compiler_params=pltpu.CompilerParams(needs_layout_passes=False),
        out_type=jax.ShapeDtypeStruct((B, H, W), jnp.float32),
        scratch_types=[
            pltpu.VMEM((P * 2,), jnp.int32),
            pltpu.VMEM((P,), jnp.int32),
            pltpu.VMEM((ROWS, W), jnp.float32),
        ],
    )(pts)
    return out.reshape(B, H, W, 1)


# speedup vs baseline: 9.1144x; 1.5041x over previous
import jax
import jax.numpy as jnp
from jax import lax
from jax.experimental import pallas as pl
from jax.experimental.pallas import tpu as pltpu
from jax.experimental.pallas import tpu_sc as plsc

H, W = 512, 512
B = 64
P = 8192
PBLK = P // 128
ROWS = 64
CHUNK = ROWS * W
CROWS = CHUNK // 128
NCHUNK = H // ROWS
NC, NS = 2, 16
NW = NC * NS
BPW = B // NW
GROUPS = P // 16
BROWS = H * W // 128


def _sc_body(pts_hbm, out_hbm, pts_v, idx_v, acc0, acc1, sem0, sem1):
    wid = lax.axis_index("s") * NC + lax.axis_index("c")
    ones = jnp.full((16,), 1.0, dtype=jnp.float32)
    zeros = jnp.zeros((16,), dtype=jnp.float32)
    bufs = (acc0, acc1)
    sems = (sem0, sem1)
    descs = [None, None]

    for j in range(BPW):
        b = wid * BPW + j
        pltpu.sync_copy(pts_hbm.at[pl.ds(b * 2 * PBLK, 2 * PBLK)], pts_v)

        def idx_body(g, _):
            blk = lax.shift_right_logical(g, 3) * 2
            u = (g & 7) * 16
            y = pts_v[blk, pl.ds(u, 16)]
            x = pts_v[blk + 1, pl.ds(u, 16)]
            idx_v[pl.ds(g * 16, 16)] = y * W + x
            return 0

        lax.fori_loop(0, GROUPS, idx_body, 0)

        for c in range(NCHUNK):
            pb = c & 1
            acc = bufs[pb]
            if descs[pb] is not None:
                descs[pb].wait()
            lo = c * CHUNK

            def zero_body(r, _):
                for u in range(128 // 16):
                    acc[r, pl.ds(u * 16, 16)] = zeros
                return 0

            lax.fori_loop(0, CROWS, zero_body, 0)

            def scat_body(g, _):
                v = idx_v[pl.ds(g * 16, 16)] - lo
                m = lax.bitcast_convert_type(v, jnp.uint32) < jnp.uint32(CHUNK)
                safe = jnp.where(m, v, 0)
                iy = lax.shift_right_logical(safe, 7)
                ix = safe & 127
                plsc.addupdate_scatter(acc, [iy, ix], ones, mask=m)
                return 0

            lax.fori_loop(0, GROUPS, scat_body, 0)

            descs[pb] = pltpu.async_copy(
                acc,
                out_hbm.at[pl.ds(b * BROWS + c * CROWS, CROWS)],
                sems[pb],
            )

    for d in descs:
        if d is not None:
            d.wait()


def kernel(points):
    pts = points.reshape(B, PBLK, 128, 2).transpose(0, 1, 3, 2)
    pts = pts.reshape(B * PBLK * 2, 128)
    mesh = plsc.VectorSubcoreMesh(core_axis_name="c", subcore_axis_name="s")
    out = pl.kernel(
        _sc_body,
        mesh=mesh,
        compiler_params=pltpu.CompilerParams(needs_layout_passes=False),
        out_type=jax.ShapeDtypeStruct((B * BROWS, 128), jnp.float32),
        scratch_types=[
            pltpu.VMEM((2 * PBLK, 128), jnp.int32),
            pltpu.VMEM((P,), jnp.int32),
            pltpu.VMEM((CROWS, 128), jnp.float32),
            pltpu.VMEM((CROWS, 128), jnp.float32),
            pltpu.SemaphoreType.DMA,
            pltpu.SemaphoreType.DMA,
        ],
    )(pts)
    return out.reshape(B, H, W, 1)


# speedup vs baseline: 9.5479x; 1.0476x over previous
import jax
import jax.numpy as jnp
from jax import lax
from jax.experimental import pallas as pl
from jax.experimental.pallas import tpu as pltpu
from jax.experimental.pallas import tpu_sc as plsc

H, W = 512, 512
B = 64
P = 8192
PBLK = P // 128
ROWS = 64
CHUNK = ROWS * W
CROWS = CHUNK // 128
NCHUNK = H // ROWS
NC, NS = 2, 16
NW = NC * NS
BPW = B // NW
GROUPS = P // 16
BROWS = H * W // 128


def _sc_body(pts_hbm, out_hbm, pts_v, idx_v, acc0, acc1, sem0, sem1):
    wid = lax.axis_index("s") * NC + lax.axis_index("c")
    ones = jnp.full((16,), 1.0, dtype=jnp.float32)
    zeros = jnp.zeros((16,), dtype=jnp.float32)
    bufs = (acc0, acc1)
    sems = (sem0, sem1)
    descs = [None, None]

    for j in range(BPW):
        b = wid * BPW + j
        pltpu.sync_copy(pts_hbm.at[pl.ds(b * 2 * PBLK, 2 * PBLK)], pts_v)

        def idx_body(blk2, _):
            blk = blk2 * 2
            for u in range(8):
                y = pts_v[blk, pl.ds(u * 16, 16)]
                x = pts_v[blk + 1, pl.ds(u * 16, 16)]
                idx_v[pl.ds((blk2 * 8 + u) * 16, 16)] = y * W + x
            return 0

        lax.fori_loop(0, PBLK, idx_body, 0)

        for c in range(NCHUNK):
            pb = c & 1
            acc = bufs[pb]
            if descs[pb] is not None:
                descs[pb].wait()
            lo = c * CHUNK

            def zero_body(r, _):
                for u in range(128 // 16):
                    acc[r, pl.ds(u * 16, 16)] = zeros
                return 0

            lax.fori_loop(0, CROWS, zero_body, 0)

            def scat_body(g4, _):
                for u in range(4):
                    v = idx_v[pl.ds((g4 * 4 + u) * 16, 16)] - lo
                    m = lax.bitcast_convert_type(v, jnp.uint32) < jnp.uint32(
                        CHUNK
                    )
                    safe = jnp.where(m, v, 0)
                    iy = lax.shift_right_logical(safe, 7)
                    ix = safe & 127
                    plsc.addupdate_scatter(acc, [iy, ix], ones, mask=m)
                return 0

            lax.fori_loop(0, GROUPS // 4, scat_body, 0)

            descs[pb] = pltpu.async_copy(
                acc,
                out_hbm.at[pl.ds(b * BROWS + c * CROWS, CROWS)],
                sems[pb],
            )

    for d in descs:
        if d is not None:
            d.wait()


def kernel(points):
    pts = points.reshape(B, PBLK, 128, 2).transpose(0, 1, 3, 2)
    pts = pts.reshape(B * PBLK * 2, 128)
    mesh = plsc.VectorSubcoreMesh(core_axis_name="c", subcore_axis_name="s")
    out = pl.kernel(
        _sc_body,
        mesh=mesh,
        compiler_params=pltpu.CompilerParams(needs_layout_passes=False),
        out_type=jax.ShapeDtypeStruct((B * BROWS, 128), jnp.float32),
        scratch_types=[
            pltpu.VMEM((2 * PBLK, 128), jnp.int32),
            pltpu.VMEM((P,), jnp.int32),
            pltpu.VMEM((CROWS, 128), jnp.float32),
            pltpu.VMEM((CROWS, 128), jnp.float32),
            pltpu.SemaphoreType.DMA,
            pltpu.SemaphoreType.DMA,
        ],
    )(pts)
    return out.reshape(B, H, W, 1)


# speedup vs baseline: 11.1570x; 1.1685x over previous
import jax
import jax.numpy as jnp
from jax import lax
from jax.experimental import pallas as pl
from jax.experimental.pallas import tpu as pltpu
from jax.experimental.pallas import tpu_sc as plsc

H, W = 512, 512
B = 64
P = 8192
PBLK = P // 128
ROWS = 64
CHUNK = ROWS * W
CROWS = CHUNK // 128
NCHUNK = H // ROWS
NC, NS = 2, 16
NW = NC * NS
BPW = B // NW
GROUPS = P // 16
BROWS = H * W // 128
NBUF = 3


def _sc_body(pts_hbm, out_hbm, pts_v, idx_v, acc0, acc1, acc2, zsh,
             zs0, zs1, zs2, os0, os1, os2):
    wid = lax.axis_index("s") * NC + lax.axis_index("c")
    sid = lax.axis_index("s")
    ones = jnp.full((16,), 1.0, dtype=jnp.float32)
    zeros = jnp.zeros((16,), dtype=jnp.float32)
    bufs = (acc0, acc1, acc2)
    zsems = (zs0, zs1, zs2)
    osems = (os0, os1, os2)
    zdescs = [None] * NBUF
    odescs = [None] * NBUF

    @pl.when(sid == 0)
    def _():
        def zb(r, _):
            for u in range(128 // 16):
                acc0[r, pl.ds(u * 16, 16)] = zeros
            return 0

        lax.fori_loop(0, CROWS, zb, 0)
        pltpu.sync_copy(acc0, zsh)

    plsc.subcore_barrier()

    for pb in range(NBUF):
        zdescs[pb] = pltpu.async_copy(zsh, bufs[pb], zsems[pb])

    for j in range(BPW):
        b = wid * BPW + j
        pltpu.sync_copy(pts_hbm.at[pl.ds(b * 2 * PBLK, 2 * PBLK)], pts_v)

        def idx_body(blk2, _):
            blk = blk2 * 2
            for u in range(8):
                y = pts_v[blk, pl.ds(u * 16, 16)]
                x = pts_v[blk + 1, pl.ds(u * 16, 16)]
                idx_v[pl.ds((blk2 * 8 + u) * 16, 16)] = y * W + x
            return 0

        lax.fori_loop(0, PBLK, idx_body, 0)

        for c in range(NCHUNK):
            cg = j * NCHUNK + c
            pb = cg % NBUF
            acc = bufs[pb]
            zdescs[pb].wait()
            lo = c * CHUNK

            def scat_body(g4, _):
                for u in range(4):
                    v = idx_v[pl.ds((g4 * 4 + u) * 16, 16)] - lo
                    m = lax.bitcast_convert_type(v, jnp.uint32) < jnp.uint32(
                        CHUNK
                    )
                    safe = jnp.where(m, v, 0)
                    iy = lax.shift_right_logical(safe, 7)
                    ix = safe & 127
                    plsc.addupdate_scatter(acc, [iy, ix], ones, mask=m)
                return 0

            lax.fori_loop(0, GROUPS // 4, scat_body, 0)

            odescs[pb] = pltpu.async_copy(
                acc,
                out_hbm.at[pl.ds(b * BROWS + c * CROWS, CROWS)],
                osems[pb],
            )

            npb = (cg + 2) % NBUF
            if cg + 2 < BPW * NCHUNK and cg >= 1:
                odescs[npb].wait()
                zdescs[npb] = pltpu.async_copy(zsh, bufs[npb], zsems[npb])

    for pb in range(NBUF):
        if odescs[pb] is not None:
            odescs[pb].wait()


def kernel(points):
    pts = points.reshape(B, PBLK, 128, 2).transpose(0, 1, 3, 2)
    pts = pts.reshape(B * PBLK * 2, 128)
    mesh = plsc.VectorSubcoreMesh(core_axis_name="c", subcore_axis_name="s")
    out = pl.kernel(
        _sc_body,
        mesh=mesh,
        compiler_params=pltpu.CompilerParams(needs_layout_passes=False),
        out_type=jax.ShapeDtypeStruct((B * BROWS, 128), jnp.float32),
        scratch_types=[
            pltpu.VMEM((2 * PBLK, 128), jnp.int32),
            pltpu.VMEM((P,), jnp.int32),
            pltpu.VMEM((CROWS, 128), jnp.float32),
            pltpu.VMEM((CROWS, 128), jnp.float32),
            pltpu.VMEM((CROWS, 128), jnp.float32),
            pltpu.VMEM_SHARED((CROWS, 128), jnp.float32),
            pltpu.SemaphoreType.DMA,
            pltpu.SemaphoreType.DMA,
            pltpu.SemaphoreType.DMA,
            pltpu.SemaphoreType.DMA,
            pltpu.SemaphoreType.DMA,
            pltpu.SemaphoreType.DMA,
        ],
    )(pts)
    return out.reshape(B, H, W, 1)


# speedup vs baseline: 13.0505x; 1.1697x over previous
import jax
import jax.numpy as jnp
from jax import lax
from jax.experimental import pallas as pl
from jax.experimental.pallas import tpu as pltpu
from jax.experimental.pallas import tpu_sc as plsc

H, W = 512, 512
B = 64
P = 8192
PBLK = P // 128
ROWS = 64
CHUNK = ROWS * W
CROWS = CHUNK // 128
NCHUNK = H // ROWS
NBKT = 4
QUART = H * W // NBKT
BCAP = P + 16
NC, NS = 2, 16
NW = NC * NS
BPW = B // NW
GROUPS = P // 16
BROWS = H * W // 128
NBUF = 2


def _sc_body(pts_hbm, out_hbm, pts_v, bk0, bk1, bk2, bk3, acc0, acc1, zsh,
             zs0, zs1, os0, os1):
    wid = lax.axis_index("s") * NC + lax.axis_index("c")
    sid = lax.axis_index("s")
    lanes = lax.iota(jnp.int32, 16)
    ones = jnp.full((16,), 1.0, dtype=jnp.float32)
    zeros = jnp.zeros((16,), dtype=jnp.float32)
    bkts = (bk0, bk1, bk2, bk3)
    bufs = (acc0, acc1)
    zsems = (zs0, zs1)
    osems = (os0, os1)
    zdescs = [None] * NBUF
    odescs = [None] * NBUF

    @pl.when(sid == 0)
    def _():
        def zb(r, _):
            for u in range(128 // 16):
                acc0[r, pl.ds(u * 16, 16)] = zeros
            return 0

        lax.fori_loop(0, CROWS, zb, 0)
        pltpu.sync_copy(acc0, zsh)

    plsc.subcore_barrier()

    for pb in range(NBUF):
        zdescs[pb] = pltpu.async_copy(zsh, bufs[pb], zsems[pb])

    for j in range(BPW):
        b = wid * BPW + j
        pltpu.sync_copy(pts_hbm.at[pl.ds(b * 2 * PBLK, 2 * PBLK)], pts_v)

        def part_body(g, offs):
            o0, o1, o2, o3 = offs
            blk = lax.shift_right_logical(g, 3) * 2
            u = (g & 7) * 16
            y = pts_v[blk, pl.ds(u, 16)]
            x = pts_v[blk + 1, pl.ds(u, 16)]
            v = y * W + x
            q = lax.shift_right_logical(v, 16)
            m0 = q == 0
            m1 = q == 1
            m2 = q == 2
            m3 = q == 3
            plsc.store_compressed(bk0.at[pl.ds(o0, 16)], v, mask=m0)
            plsc.store_compressed(bk1.at[pl.ds(o1, 16)], v, mask=m1)
            plsc.store_compressed(bk2.at[pl.ds(o2, 16)], v, mask=m2)
            plsc.store_compressed(bk3.at[pl.ds(o3, 16)], v, mask=m3)
            c0 = jnp.sum(m0.astype(jnp.int32))
            c1 = jnp.sum(m1.astype(jnp.int32))
            c2 = jnp.sum(m2.astype(jnp.int32))
            no0 = o0 + c0
            no1 = o1 + c1
            no2 = o2 + c2
            no3 = (g + 1) * 16 - no0 - no1 - no2
            return (no0, no1, no2, no3)

        z32 = jnp.int32(0)
        n0, n1, n2, n3 = lax.fori_loop(
            0, GROUPS, part_body, (z32, z32, z32, z32)
        )
        ns = (n0, n1, n2, n3)

        for c in range(NCHUNK):
            cg = j * NCHUNK + c
            pb = cg % NBUF
            acc = bufs[pb]
            bk = bkts[c // (NCHUNK // NBKT)]
            nk = ns[c // (NCHUNK // NBKT)]
            zdescs[pb].wait()
            lo = c * CHUNK

            def scat_body(g, _):
                v = bk[pl.ds(g * 16, 16)] - lo
                m = lax.bitcast_convert_type(v, jnp.uint32) < jnp.uint32(
                    CHUNK
                )
                m = m & (g * 16 + lanes < nk)
                safe = jnp.where(m, v, 0)
                iy = lax.shift_right_logical(safe, 7)
                ix = safe & 127
                plsc.addupdate_scatter(acc, [iy, ix], ones, mask=m)
                return 0

            niter = lax.shift_right_logical(nk + 15, 4)
            lax.fori_loop(0, niter, scat_body, 0)

            odescs[pb] = pltpu.async_copy(
                acc,
                out_hbm.at[pl.ds(b * BROWS + c * CROWS, CROWS)],
                osems[pb],
            )

            if cg >= 1 and cg + 1 < BPW * NCHUNK:
                npb = (cg + 1) % NBUF
                odescs[npb].wait()
                zdescs[npb] = pltpu.async_copy(zsh, bufs[npb], zsems[npb])

    for pb in range(NBUF):
        if odescs[pb] is not None:
            odescs[pb].wait()


def kernel(points):
    pts = points.reshape(B, PBLK, 128, 2).transpose(0, 1, 3, 2)
    pts = pts.reshape(B * PBLK * 2, 128)
    mesh = plsc.VectorSubcoreMesh(core_axis_name="c", subcore_axis_name="s")
    out = pl.kernel(
        _sc_body,
        mesh=mesh,
        compiler_params=pltpu.CompilerParams(needs_layout_passes=False),
        out_type=jax.ShapeDtypeStruct((B * BROWS, 128), jnp.float32),
        scratch_types=[
            pltpu.VMEM((2 * PBLK, 128), jnp.int32),
            pltpu.VMEM((BCAP,), jnp.int32),
            pltpu.VMEM((BCAP,), jnp.int32),
            pltpu.VMEM((BCAP,), jnp.int32),
            pltpu.VMEM((BCAP,), jnp.int32),
            pltpu.VMEM((CROWS, 128), jnp.float32),
            pltpu.VMEM((CROWS, 128), jnp.float32),
            pltpu.VMEM_SHARED((CROWS, 128), jnp.float32),
            pltpu.SemaphoreType.DMA,
            pltpu.SemaphoreType.DMA,
            pltpu.SemaphoreType.DMA,
            pltpu.SemaphoreType.DMA,
        ],
    )(pts)
    return out.reshape(B, H, W, 1)


# speedup vs baseline: 13.3732x; 1.0247x over previous
import jax
import jax.numpy as jnp
from jax import lax
from jax.experimental import pallas as pl
from jax.experimental.pallas import tpu as pltpu
from jax.experimental.pallas import tpu_sc as plsc

H, W = 512, 512
B = 64
P = 8192
PBLK = P // 128
ROWS = 64
CHUNK = ROWS * W
CROWS = CHUNK // 128
NCHUNK = H // ROWS
NBKT = 4
QUART = H * W // NBKT
BCAP = P + 16
NC, NS = 2, 16
NW = NC * NS
BPW = B // NW
GROUPS = P // 16
BROWS = H * W // 128
NBUF = 2


def _sc_body(pts_hbm, out_hbm, pts_v, bk0, bk1, bk2, bk3, acc0, acc1, zsh,
             zs0, zs1, os0, os1):
    wid = lax.axis_index("s") * NC + lax.axis_index("c")
    sid = lax.axis_index("s")
    lanes = lax.iota(jnp.int32, 16)
    ones = jnp.full((16,), 1.0, dtype=jnp.float32)
    zeros = jnp.zeros((16,), dtype=jnp.float32)
    bkts = (bk0, bk1, bk2, bk3)
    bufs = (acc0, acc1)
    zsems = (zs0, zs1)
    osems = (os0, os1)
    zdescs = [None] * NBUF
    odescs = [None] * NBUF

    @pl.when(sid == 0)
    def _():
        def zb(r, _):
            for u in range(128 // 16):
                acc0[r, pl.ds(u * 16, 16)] = zeros
            return 0

        lax.fori_loop(0, CROWS, zb, 0)
        pltpu.sync_copy(acc0, zsh)

    plsc.subcore_barrier()

    for pb in range(NBUF):
        zdescs[pb] = pltpu.async_copy(zsh, bufs[pb], zsems[pb])

    for j in range(BPW):
        b = wid * BPW + j
        pltpu.sync_copy(pts_hbm.at[pl.ds(b * 2 * PBLK, 2 * PBLK)], pts_v)

        def part_body(g, offs):
            o0, o1, o2, o3 = offs
            blk = lax.shift_right_logical(g, 3) * 2
            u = (g & 7) * 16
            y = pts_v[blk, pl.ds(u, 16)]
            x = pts_v[blk + 1, pl.ds(u, 16)]
            v = y * W + x
            q = lax.shift_right_logical(v, 16)
            m0 = q == 0
            m1 = q == 1
            m2 = q == 2
            m3 = q == 3
            plsc.store_compressed(bk0.at[pl.ds(o0, 16)], v, mask=m0)
            plsc.store_compressed(bk1.at[pl.ds(o1, 16)], v, mask=m1)
            plsc.store_compressed(bk2.at[pl.ds(o2, 16)], v, mask=m2)
            plsc.store_compressed(bk3.at[pl.ds(o3, 16)], v, mask=m3)
            c0 = plsc.all_reduce_population_count(m0)[0]
            c1 = plsc.all_reduce_population_count(m1)[0]
            c2 = plsc.all_reduce_population_count(m2)[0]
            no0 = o0 + c0
            no1 = o1 + c1
            no2 = o2 + c2
            no3 = (g + 1) * 16 - no0 - no1 - no2
            return (no0, no1, no2, no3)

        z32 = jnp.int32(0)
        n0, n1, n2, n3 = lax.fori_loop(
            0, GROUPS, part_body, (z32, z32, z32, z32)
        )
        ns = (n0, n1, n2, n3)

        for c in range(NCHUNK):
            cg = j * NCHUNK + c
            pb = cg % NBUF
            acc = bufs[pb]
            bk = bkts[c // (NCHUNK // NBKT)]
            nk = ns[c // (NCHUNK // NBKT)]
            zdescs[pb].wait()
            lo = c * CHUNK

            def scat_body(g, _):
                v = bk[pl.ds(g * 16, 16)] - lo
                m = lax.bitcast_convert_type(v, jnp.uint32) < jnp.uint32(
                    CHUNK
                )
                m = m & (g * 16 + lanes < nk)
                safe = jnp.where(m, v, 0)
                iy = lax.shift_right_logical(safe, 7)
                ix = safe & 127
                plsc.addupdate_scatter(acc, [iy, ix], ones, mask=m)
                return 0

            niter = lax.shift_right_logical(nk + 15, 4)
            lax.fori_loop(0, niter, scat_body, 0)

            odescs[pb] = pltpu.async_copy(
                acc,
                out_hbm.at[pl.ds(b * BROWS + c * CROWS, CROWS)],
                osems[pb],
            )

            if cg >= 1 and cg + 1 < BPW * NCHUNK:
                npb = (cg + 1) % NBUF
                odescs[npb].wait()
                zdescs[npb] = pltpu.async_copy(zsh, bufs[npb], zsems[npb])

    for pb in range(NBUF):
        if odescs[pb] is not None:
            odescs[pb].wait()


def kernel(points):
    pts = points.reshape(B, PBLK, 128, 2).transpose(0, 1, 3, 2)
    pts = pts.reshape(B * PBLK * 2, 128)
    mesh = plsc.VectorSubcoreMesh(core_axis_name="c", subcore_axis_name="s")
    out = pl.kernel(
        _sc_body,
        mesh=mesh,
        compiler_params=pltpu.CompilerParams(needs_layout_passes=False),
        out_type=jax.ShapeDtypeStruct((B * BROWS, 128), jnp.float32),
        scratch_types=[
            pltpu.VMEM((2 * PBLK, 128), jnp.int32),
            pltpu.VMEM((BCAP,), jnp.int32),
            pltpu.VMEM((BCAP,), jnp.int32),
            pltpu.VMEM((BCAP,), jnp.int32),
            pltpu.VMEM((BCAP,), jnp.int32),
            pltpu.VMEM((CROWS, 128), jnp.float32),
            pltpu.VMEM((CROWS, 128), jnp.float32),
            pltpu.VMEM_SHARED((CROWS, 128), jnp.float32),
            pltpu.SemaphoreType.DMA,
            pltpu.SemaphoreType.DMA,
            pltpu.SemaphoreType.DMA,
            pltpu.SemaphoreType.DMA,
        ],
    )(pts)
    return out.reshape(B, H, W, 1)


# speedup vs baseline: 13.7801x; 1.0304x over previous
import jax
import jax.numpy as jnp
from jax import lax
from jax.experimental import pallas as pl
from jax.experimental.pallas import tpu as pltpu
from jax.experimental.pallas import tpu_sc as plsc

H, W = 512, 512
B = 64
P = 8192
PBLK = P // 128
ROWS = 32
CHUNK = ROWS * W
CROWS = CHUNK // 128
NCHUNK = H // ROWS
NBKT = 4
QUART = H * W // NBKT
BCAP = P + 16
NC, NS = 2, 16
NW = NC * NS
BPW = B // NW
GROUPS = P // 16
BROWS = H * W // 128
NBUF = 3


def _sc_body(pts_hbm, out_hbm, pts_v, bk0, bk1, bk2, bk3, acc0, acc1, acc2,
             zsh, zs0, zs1, zs2, os0, os1, os2):
    wid = lax.axis_index("s") * NC + lax.axis_index("c")
    sid = lax.axis_index("s")
    lanes = lax.iota(jnp.int32, 16)
    ones = jnp.full((16,), 1.0, dtype=jnp.float32)
    zeros = jnp.zeros((16,), dtype=jnp.float32)
    bkts = (bk0, bk1, bk2, bk3)
    bufs = (acc0, acc1, acc2)
    zsems = (zs0, zs1, zs2)
    osems = (os0, os1, os2)
    zdescs = [None] * NBUF
    odescs = [None] * NBUF

    @pl.when(sid == 0)
    def _():
        def zb(r, _):
            for u in range(128 // 16):
                acc0[r, pl.ds(u * 16, 16)] = zeros
            return 0

        lax.fori_loop(0, CROWS, zb, 0)
        pltpu.sync_copy(acc0, zsh)

    plsc.subcore_barrier()

    for pb in range(NBUF):
        zdescs[pb] = pltpu.async_copy(zsh, bufs[pb], zsems[pb])

    for j in range(BPW):
        b = wid * BPW + j
        pltpu.sync_copy(pts_hbm.at[pl.ds(b * 2 * PBLK, 2 * PBLK)], pts_v)

        def part_body(g, offs):
            o0, o1, o2, o3 = offs
            blk = lax.shift_right_logical(g, 3) * 2
            u = (g & 7) * 16
            y = pts_v[blk, pl.ds(u, 16)]
            x = pts_v[blk + 1, pl.ds(u, 16)]
            v = y * W + x
            q = lax.shift_right_logical(v, 16)
            m0 = q == 0
            m1 = q == 1
            m2 = q == 2
            m3 = q == 3
            plsc.store_compressed(bk0.at[pl.ds(o0, 16)], v, mask=m0)
            plsc.store_compressed(bk1.at[pl.ds(o1, 16)], v, mask=m1)
            plsc.store_compressed(bk2.at[pl.ds(o2, 16)], v, mask=m2)
            plsc.store_compressed(bk3.at[pl.ds(o3, 16)], v, mask=m3)
            c0 = plsc.all_reduce_population_count(m0)[0]
            c1 = plsc.all_reduce_population_count(m1)[0]
            c2 = plsc.all_reduce_population_count(m2)[0]
            no0 = o0 + c0
            no1 = o1 + c1
            no2 = o2 + c2
            no3 = (g + 1) * 16 - no0 - no1 - no2
            return (no0, no1, no2, no3)

        z32 = jnp.int32(0)
        n0, n1, n2, n3 = lax.fori_loop(
            0, GROUPS, part_body, (z32, z32, z32, z32)
        )
        ns = (n0, n1, n2, n3)

        for c in range(NCHUNK):
            cg = j * NCHUNK + c
            pb = cg % NBUF
            acc = bufs[pb]
            bk = bkts[c // (NCHUNK // NBKT)]
            nk = ns[c // (NCHUNK // NBKT)]
            zdescs[pb].wait()
            lo = c * CHUNK

            def scat_body(g, _):
                v = bk[pl.ds(g * 16, 16)] - lo
                m = lax.bitcast_convert_type(v, jnp.uint32) < jnp.uint32(
                    CHUNK
                )
                m = m & (g * 16 + lanes < nk)
                safe = jnp.where(m, v, 0)
                iy = lax.shift_right_logical(safe, 7)
                ix = safe & 127
                plsc.addupdate_scatter(acc, [iy, ix], ones, mask=m)
                return 0

            niter = lax.shift_right_logical(nk + 15, 4)
            lax.fori_loop(0, niter, scat_body, 0)

            odescs[pb] = pltpu.async_copy(
                acc,
                out_hbm.at[pl.ds(b * BROWS + c * CROWS, CROWS)],
                osems[pb],
            )

            if cg >= 1 and cg + 2 < BPW * NCHUNK:
                npb = (cg + 2) % NBUF
                odescs[npb].wait()
                zdescs[npb] = pltpu.async_copy(zsh, bufs[npb], zsems[npb])

    for pb in range(NBUF):
        if odescs[pb] is not None:
            odescs[pb].wait()


def kernel(points):
    pts = points.reshape(B, PBLK, 128, 2).transpose(0, 1, 3, 2)
    pts = pts.reshape(B * PBLK * 2, 128)
    mesh = plsc.VectorSubcoreMesh(core_axis_name="c", subcore_axis_name="s")
    out = pl.kernel(
        _sc_body,
        mesh=mesh,
        compiler_params=pltpu.CompilerParams(needs_layout_passes=False),
        out_type=jax.ShapeDtypeStruct((B * BROWS, 128), jnp.float32),
        scratch_types=[
            pltpu.VMEM((2 * PBLK, 128), jnp.int32),
            pltpu.VMEM((BCAP,), jnp.int32),
            pltpu.VMEM((BCAP,), jnp.int32),
            pltpu.VMEM((BCAP,), jnp.int32),
            pltpu.VMEM((BCAP,), jnp.int32),
            pltpu.VMEM((CROWS, 128), jnp.float32),
            pltpu.VMEM((CROWS, 128), jnp.float32),
            pltpu.VMEM((CROWS, 128), jnp.float32),
            pltpu.VMEM_SHARED((CROWS, 128), jnp.float32),
            pltpu.SemaphoreType.DMA,
            pltpu.SemaphoreType.DMA,
            pltpu.SemaphoreType.DMA,
            pltpu.SemaphoreType.DMA,
            pltpu.SemaphoreType.DMA,
            pltpu.SemaphoreType.DMA,
        ],
    )(pts)
    return out.reshape(B, H, W, 1)


# speedup vs baseline: 14.1129x; 1.0241x over previous
import jax
import jax.numpy as jnp
from jax import lax
from jax.experimental import pallas as pl
from jax.experimental.pallas import tpu as pltpu
from jax.experimental.pallas import tpu_sc as plsc

H, W = 512, 512
B = 64
P = 8192
PBLK = P // 128
ROWS = 32
CHUNK = ROWS * W
CROWS = CHUNK // 128
NCHUNK = H // ROWS
NBKT = 4
QUART = H * W // NBKT
BCAP = P + 16
NC, NS = 2, 16
NW = NC * NS
BPW = B // NW
GROUPS = P // 16
BROWS = H * W // 128
NBUF = 3


def _sc_body(pts_hbm, out_hbm, pts_v, bk0, bk1, bk2, bk3, acc0, acc1, acc2,
             zsh, zs0, zs1, zs2, os0, os1, os2):
    wid = lax.axis_index("s") * NC + lax.axis_index("c")
    sid = lax.axis_index("s")
    lanes = lax.iota(jnp.int32, 16)
    ones = jnp.full((16,), 1.0, dtype=jnp.float32)
    zeros = jnp.zeros((16,), dtype=jnp.float32)
    bkts = (bk0, bk1, bk2, bk3)
    bufs = (acc0, acc1, acc2)
    zsems = (zs0, zs1, zs2)
    osems = (os0, os1, os2)
    zdescs = [None] * NBUF
    odescs = [None] * NBUF

    @pl.when(sid == 0)
    def _():
        def zb(r, _):
            for u in range(128 // 16):
                acc0[r, pl.ds(u * 16, 16)] = zeros
            return 0

        lax.fori_loop(0, CROWS, zb, 0)
        pltpu.sync_copy(acc0, zsh)

    plsc.subcore_barrier()

    for pb in range(NBUF):
        zdescs[pb] = pltpu.async_copy(zsh, bufs[pb], zsems[pb])

    for j in range(BPW):
        b = wid * BPW + j
        pltpu.sync_copy(pts_hbm.at[pl.ds(b * 2 * PBLK, 2 * PBLK)], pts_v)

        def part_body(g, offs):
            o0, o1, o2, o3 = offs
            blk = lax.shift_right_logical(g, 3) * 2
            u = (g & 7) * 16
            y = pts_v[blk, pl.ds(u, 16)]
            x = pts_v[blk + 1, pl.ds(u, 16)]
            v = y * W + x
            q = lax.shift_right_logical(v, 16)
            m0 = q == 0
            m1 = q == 1
            m2 = q == 2
            m3 = q == 3
            plsc.store_compressed(bk0.at[pl.ds(o0, 16)], v, mask=m0)
            plsc.store_compressed(bk1.at[pl.ds(o1, 16)], v, mask=m1)
            plsc.store_compressed(bk2.at[pl.ds(o2, 16)], v, mask=m2)
            plsc.store_compressed(bk3.at[pl.ds(o3, 16)], v, mask=m3)
            c0 = plsc.all_reduce_population_count(m0)[0]
            c1 = plsc.all_reduce_population_count(m1)[0]
            c2 = plsc.all_reduce_population_count(m2)[0]
            no0 = o0 + c0
            no1 = o1 + c1
            no2 = o2 + c2
            no3 = (g + 1) * 16 - no0 - no1 - no2
            return (no0, no1, no2, no3)

        z32 = jnp.int32(0)
        n0, n1, n2, n3 = lax.fori_loop(
            0, GROUPS, part_body, (z32, z32, z32, z32)
        )
        ns = (n0, n1, n2, n3)

        for c in range(NCHUNK):
            cg = j * NCHUNK + c
            pb = cg % NBUF
            acc = bufs[pb]
            bk = bkts[c // (NCHUNK // NBKT)]
            nk = ns[c // (NCHUNK // NBKT)]
            zdescs[pb].wait()
            lo = c * CHUNK

            def scat_body(g2, _):
                for u in range(2):
                    g = g2 * 2 + u
                    v = bk[pl.ds(g * 16, 16)] - lo
                    m = lax.bitcast_convert_type(
                        v, jnp.uint32
                    ) < jnp.uint32(CHUNK)
                    m = m & (g * 16 + lanes < nk)
                    safe = jnp.where(m, v, 0)
                    iy = lax.shift_right_logical(safe, 7)
                    ix = safe & 127
                    plsc.addupdate_scatter(acc, [iy, ix], ones, mask=m)
                return 0

            niter = lax.shift_right_logical(nk + 31, 5)
            lax.fori_loop(0, niter, scat_body, 0)

            odescs[pb] = pltpu.async_copy(
                acc,
                out_hbm.at[pl.ds(b * BROWS + c * CROWS, CROWS)],
                osems[pb],
            )

            if cg >= 1 and cg + 2 < BPW * NCHUNK:
                npb = (cg + 2) % NBUF
                odescs[npb].wait()
                zdescs[npb] = pltpu.async_copy(zsh, bufs[npb], zsems[npb])

    for pb in range(NBUF):
        if odescs[pb] is not None:
            odescs[pb].wait()


def kernel(points):
    pts = points.reshape(B, PBLK, 128, 2).transpose(0, 1, 3, 2)
    pts = pts.reshape(B * PBLK * 2, 128)
    mesh = plsc.VectorSubcoreMesh(core_axis_name="c", subcore_axis_name="s")
    out = pl.kernel(
        _sc_body,
        mesh=mesh,
        compiler_params=pltpu.CompilerParams(needs_layout_passes=False),
        out_type=jax.ShapeDtypeStruct((B * BROWS, 128), jnp.float32),
        scratch_types=[
            pltpu.VMEM((2 * PBLK, 128), jnp.int32),
            pltpu.VMEM((BCAP,), jnp.int32),
            pltpu.VMEM((BCAP,), jnp.int32),
            pltpu.VMEM((BCAP,), jnp.int32),
            pltpu.VMEM((BCAP,), jnp.int32),
            pltpu.VMEM((CROWS, 128), jnp.float32),
            pltpu.VMEM((CROWS, 128), jnp.float32),
            pltpu.VMEM((CROWS, 128), jnp.float32),
            pltpu.VMEM_SHARED((CROWS, 128), jnp.float32),
            pltpu.SemaphoreType.DMA,
            pltpu.SemaphoreType.DMA,
            pltpu.SemaphoreType.DMA,
            pltpu.SemaphoreType.DMA,
            pltpu.SemaphoreType.DMA,
            pltpu.SemaphoreType.DMA,
        ],
    )(pts)
    return out.reshape(B, H, W, 1)
